# Initial kernel scaffold; baseline (speedup 1.0000x reference)
import jax
import jax.numpy as jnp
from jax.experimental import pallas as pl


def kernel(embeddings, signal_edges, random_pairs):
    raise NotImplementedError("write your pallas kernel here")


# speedup vs baseline: 8.2488x; 8.2488x over previous
import functools

import jax
import jax.numpy as jnp
from jax import lax
from jax.experimental import pallas as pl
from jax.experimental.pallas import tpu as pltpu
from jax.experimental.pallas import tpu_sc as plsc

MARGIN = 1.0
K_NN = 8
N_NODES = 10000
D_EMB = 32


ROWS_PER_STEP = 80


def _knn_body(xq_ref, x_ref, out_ref):
    i = pl.program_id(0)
    q = xq_ref[...]
    x = x_ref[...]
    qx = lax.dot_general(q, x, (((1,), (1,)), ((), ())),
                         preferred_element_type=jnp.float32)
    q2 = jnp.sum(q * q, axis=1, keepdims=True)
    x2 = jnp.sum(x * x, axis=1)[None, :]
    d2 = q2 + x2 - 2.0 * qx

    R = ROWS_PER_STEP
    row = i * R + lax.broadcasted_iota(jnp.int32, (R, N_NODES), 0)
    col = lax.broadcasted_iota(jnp.int32, (R, N_NODES), 1)
    big = jnp.float32(jnp.inf)
    d2 = jnp.where(col == row, big, d2)

    remaining = jnp.full((R, 1), float(K_NN), jnp.float32)
    acc = jnp.float32(0.0)
    for _ in range(K_NN):
        m = jnp.min(d2, axis=1, keepdims=True)
        eq = d2 == m
        cnt = jnp.sum(eq.astype(jnp.float32), axis=1, keepdims=True)
        w = jnp.minimum(cnt, remaining)
        remaining = remaining - w
        acc = acc + jnp.sum(w * jnp.maximum(0.0, MARGIN - m))
        d2 = jnp.where(eq, big, d2)
    out_ref[0, 0, 0] = acc


def _knn_hinge_sum(x):
    grid = N_NODES // ROWS_PER_STEP
    partials = pl.pallas_call(
        _knn_body,
        grid=(grid,),
        in_specs=[
            pl.BlockSpec((ROWS_PER_STEP, D_EMB), lambda i: (i, 0)),
            pl.BlockSpec((N_NODES, D_EMB), lambda i: (0, 0)),
        ],
        out_specs=pl.BlockSpec((1, 1, 1), lambda i: (i, 0, 0),
                               memory_space=pltpu.SMEM),
        out_shape=jax.ShapeDtypeStruct((grid, 1, 1), jnp.float32),
    )(x, x)
    return jnp.sum(partials)


NUM_WORKERS = 32
LANES = 16
SLAB = 128
SLABS_PER_CHUNK = 5
CHUNK = SLAB * SLABS_PER_CHUNK
UNROLL = 8

SIG_PAD = 163840
RND_PAD = 102400
SIG_CHUNKS = SIG_PAD // (NUM_WORKERS * CHUNK)
RND_CHUNKS = RND_PAD // (NUM_WORKERS * CHUNK)
RND_PAD_CORRECTION = float(RND_PAD - 100000)


def _pair_kernel_body(table, sig_s, sig_d, rnd_s, rnd_d, out,
                      idx_s, idx_d, rows_s, rows_d, obuf, sem_s, sem_d):
    nc = 2
    wid = lax.axis_index("s") * nc + lax.axis_index("c")

    def accumulate(src_hbm, dst_hbm, edge_base, nchunks, hinge, acc0):
        def chunk_body(ci, acc):
            base = edge_base + ci * CHUNK
            pltpu.sync_copy(src_hbm.at[pl.ds(base, CHUNK)], idx_s)
            pltpu.sync_copy(dst_hbm.at[pl.ds(base, CHUNK)], idx_d)
            copies = []
            for j in range(SLABS_PER_CHUNK):
                copies.append(pltpu.async_copy(
                    table.at[idx_s.at[pl.ds(j * SLAB, SLAB)]],
                    rows_s.at[pl.ds(j * SLAB, SLAB)], sem_s))
                copies.append(pltpu.async_copy(
                    table.at[idx_d.at[pl.ds(j * SLAB, SLAB)]],
                    rows_d.at[pl.ds(j * SLAB, SLAB)], sem_d))
            for cp in copies:
                cp.wait()

            def group(g, a):
                for u in range(UNROLL):
                    e = g * UNROLL + u
                    rs = rows_s.at[e]
                    rd = rows_d.at[e]
                    df0 = rs[pl.ds(0, LANES)] - rd[pl.ds(0, LANES)]
                    df1 = rs[pl.ds(LANES, LANES)] - rd[pl.ds(LANES, LANES)]
                    d2 = jnp.sum(df0 * df0 + df1 * df1)
                    if hinge:
                        a = a + jnp.maximum(0.0, MARGIN - d2)
                    else:
                        a = a + d2
                return a

            return lax.fori_loop(0, CHUNK // UNROLL, group, acc)

        return lax.fori_loop(0, nchunks, chunk_body, acc0)

    acc_sig = accumulate(sig_s, sig_d, wid * (SIG_CHUNKS * CHUNK),
                         SIG_CHUNKS, False, jnp.float32(0.0))
    acc_rnd = accumulate(rnd_s, rnd_d, wid * (RND_CHUNKS * CHUNK),
                         RND_CHUNKS, True, jnp.float32(0.0))
    lane = lax.iota(jnp.int32, LANES)
    obuf[pl.ds(0, LANES)] = jnp.where(lane == 0, acc_sig, 0.0)
    obuf[pl.ds(LANES, LANES)] = jnp.where(lane == 0, acc_rnd, 0.0)
    pltpu.sync_copy(obuf, out.at[wid])


def _pair_sums(x, sig_src, sig_dst, rnd_src, rnd_dst):
    kern = functools.partial(
        pl.kernel,
        out_type=jax.ShapeDtypeStruct((NUM_WORKERS, 2 * LANES), jnp.float32),
        mesh=plsc.VectorSubcoreMesh(core_axis_name="c", subcore_axis_name="s"),
        compiler_params=pltpu.CompilerParams(
            needs_layout_passes=False, use_tc_tiling_on_sc=False),
        scratch_types=[
            pltpu.VMEM((CHUNK,), jnp.int32),
            pltpu.VMEM((CHUNK,), jnp.int32),
            pltpu.VMEM((CHUNK, D_EMB), jnp.float32),
            pltpu.VMEM((CHUNK, D_EMB), jnp.float32),
            pltpu.VMEM((2 * LANES,), jnp.float32),
            pltpu.SemaphoreType.DMA,
            pltpu.SemaphoreType.DMA,
        ],
    )(_pair_kernel_body)
    return kern(x, sig_src, sig_dst, rnd_src, rnd_dst)


def _pad_pairs(pairs, total):
    p = pairs.astype(jnp.int32)
    pad = total - p.shape[1]
    return jnp.pad(p[0], (0, pad)), jnp.pad(p[1], (0, pad))


def kernel(embeddings, signal_edges, random_pairs):
    x = embeddings.astype(jnp.float32)
    sig_src, sig_dst = _pad_pairs(signal_edges, SIG_PAD)
    rnd_src, rnd_dst = _pad_pairs(random_pairs, RND_PAD)

    knn_sum = _knn_hinge_sum(x)
    pair_out = _pair_sums(x, sig_src, sig_dst, rnd_src, rnd_dst)
    sig_sum = jnp.sum(pair_out[:, :LANES])
    rnd_sum = jnp.sum(pair_out[:, LANES:]) - RND_PAD_CORRECTION

    n_sig = signal_edges.shape[1]
    n_rnd = random_pairs.shape[1]
    return (sig_sum / n_sig + knn_sum / (N_NODES * K_NN) + rnd_sum / n_rnd
            ).astype(jnp.float32)


# speedup vs baseline: 15.3525x; 1.8612x over previous
import functools

import jax
import jax.numpy as jnp
from jax import lax
from jax.experimental import pallas as pl
from jax.experimental.pallas import tpu as pltpu
from jax.experimental.pallas import tpu_sc as plsc

MARGIN = 1.0
K_NN = 8
N_NODES = 10000
D_EMB = 32


ROWS_PER_STEP = 400


_IDX_BITS = 14


def _knn_body(xq_ref, x_ref, out_ref, x2_ref):
    i = pl.program_id(0)
    q = xq_ref[...]

    @pl.when(i == 0)
    def _():
        x = x_ref[...]
        x2_ref[...] = jnp.sum(x * x, axis=1)[None, :]

    x = x_ref[...]
    qx = lax.dot_general(q, x, (((1,), (1,)), ((), ())),
                         preferred_element_type=jnp.float32)
    q2 = jnp.sum(q * q, axis=1, keepdims=True)
    d2 = q2 + x2_ref[...] - 2.0 * qx
    d2 = jnp.abs(d2)

    idx_mask = -(1 << _IDX_BITS)
    half_ulp = 1 << (_IDX_BITS - 1)
    sentinel = 0x7F7FFFFF
    R = ROWS_PER_STEP
    row = i * R + lax.broadcasted_iota(jnp.int32, (R, N_NODES), 0)
    col = lax.broadcasted_iota(jnp.int32, (R, N_NODES), 1)
    bits = lax.bitcast_convert_type(d2, jnp.int32)
    kbits = jnp.bitwise_or(jnp.bitwise_and(bits, idx_mask), col)
    kbits = jnp.where(col == row, sentinel, kbits)
    key = lax.bitcast_convert_type(kbits, jnp.float32)

    acc = jnp.float32(0.0)
    big = lax.bitcast_convert_type(jnp.int32(sentinel), jnp.float32)
    m = jnp.full((R, 1), -1.0, jnp.float32)
    for _ in range(K_NN):
        m = jnp.min(jnp.where(key > m, key, big),
                    axis=1, keepdims=True)
        mbits = lax.bitcast_convert_type(m, jnp.int32)
        vbits = jnp.bitwise_and(mbits, idx_mask) + half_ulp
        val = lax.bitcast_convert_type(vbits, jnp.float32)
        acc = acc + jnp.sum(jnp.maximum(0.0, MARGIN - val))
    out_ref[0, 0, 0] = acc


def _knn_hinge_sum(x):
    grid = N_NODES // ROWS_PER_STEP
    partials = pl.pallas_call(
        _knn_body,
        grid=(grid,),
        in_specs=[
            pl.BlockSpec((ROWS_PER_STEP, D_EMB), lambda i: (i, 0)),
            pl.BlockSpec((N_NODES, D_EMB), lambda i: (0, 0)),
        ],
        out_specs=pl.BlockSpec((1, 1, 1), lambda i: (i, 0, 0),
                               memory_space=pltpu.SMEM),
        out_shape=jax.ShapeDtypeStruct((grid, 1, 1), jnp.float32),
        scratch_shapes=[pltpu.VMEM((1, N_NODES), jnp.float32)],
    )(x, x)
    return jnp.sum(partials)


NUM_WORKERS = 32
LANES = 16
SLAB = 128
SLABS_PER_CHUNK = 5
CHUNK = SLAB * SLABS_PER_CHUNK
UNROLL = 8

SIG_PAD = 163840
RND_PAD = 102400
SIG_CHUNKS = SIG_PAD // (NUM_WORKERS * CHUNK)
RND_CHUNKS = RND_PAD // (NUM_WORKERS * CHUNK)
RND_PAD_CORRECTION = float(RND_PAD - 100000)


def _pair_kernel_body(table, sig_s, sig_d, rnd_s, rnd_d, out,
                      idx_s, idx_d, rows_s, rows_d, obuf, sem_s, sem_d):
    nc = 2
    wid = lax.axis_index("s") * nc + lax.axis_index("c")

    def accumulate(src_hbm, dst_hbm, edge_base, nchunks, hinge, acc0):
        def chunk_body(ci, acc):
            base = edge_base + ci * CHUNK
            pltpu.sync_copy(src_hbm.at[pl.ds(base, CHUNK)], idx_s)
            pltpu.sync_copy(dst_hbm.at[pl.ds(base, CHUNK)], idx_d)
            copies = []
            for j in range(SLABS_PER_CHUNK):
                copies.append(pltpu.async_copy(
                    table.at[idx_s.at[pl.ds(j * SLAB, SLAB)]],
                    rows_s.at[pl.ds(j * SLAB, SLAB)], sem_s))
                copies.append(pltpu.async_copy(
                    table.at[idx_d.at[pl.ds(j * SLAB, SLAB)]],
                    rows_d.at[pl.ds(j * SLAB, SLAB)], sem_d))
            for cp in copies:
                cp.wait()

            def group(g, a):
                for u in range(UNROLL):
                    e = g * UNROLL + u
                    rs = rows_s.at[e]
                    rd = rows_d.at[e]
                    df0 = rs[pl.ds(0, LANES)] - rd[pl.ds(0, LANES)]
                    df1 = rs[pl.ds(LANES, LANES)] - rd[pl.ds(LANES, LANES)]
                    d2 = jnp.sum(df0 * df0 + df1 * df1)
                    if hinge:
                        a = a + jnp.maximum(0.0, MARGIN - d2)
                    else:
                        a = a + d2
                return a

            return lax.fori_loop(0, CHUNK // UNROLL, group, acc)

        return lax.fori_loop(0, nchunks, chunk_body, acc0)

    acc_sig = accumulate(sig_s, sig_d, wid * (SIG_CHUNKS * CHUNK),
                         SIG_CHUNKS, False, jnp.float32(0.0))
    acc_rnd = accumulate(rnd_s, rnd_d, wid * (RND_CHUNKS * CHUNK),
                         RND_CHUNKS, True, jnp.float32(0.0))
    lane = lax.iota(jnp.int32, LANES)
    obuf[pl.ds(0, LANES)] = jnp.where(lane == 0, acc_sig, 0.0)
    obuf[pl.ds(LANES, LANES)] = jnp.where(lane == 0, acc_rnd, 0.0)
    pltpu.sync_copy(obuf, out.at[wid])


def _pair_sums(x, sig_src, sig_dst, rnd_src, rnd_dst):
    kern = functools.partial(
        pl.kernel,
        out_type=jax.ShapeDtypeStruct((NUM_WORKERS, 2 * LANES), jnp.float32),
        mesh=plsc.VectorSubcoreMesh(core_axis_name="c", subcore_axis_name="s"),
        compiler_params=pltpu.CompilerParams(
            needs_layout_passes=False, use_tc_tiling_on_sc=False),
        scratch_types=[
            pltpu.VMEM((CHUNK,), jnp.int32),
            pltpu.VMEM((CHUNK,), jnp.int32),
            pltpu.VMEM((CHUNK, D_EMB), jnp.float32),
            pltpu.VMEM((CHUNK, D_EMB), jnp.float32),
            pltpu.VMEM((2 * LANES,), jnp.float32),
            pltpu.SemaphoreType.DMA,
            pltpu.SemaphoreType.DMA,
        ],
    )(_pair_kernel_body)
    return kern(x, sig_src, sig_dst, rnd_src, rnd_dst)


def _pad_pairs(pairs, total):
    p = pairs.astype(jnp.int32)
    pad = total - p.shape[1]
    return jnp.pad(p[0], (0, pad)), jnp.pad(p[1], (0, pad))


def kernel(embeddings, signal_edges, random_pairs):
    x = embeddings.astype(jnp.float32)
    sig_src, sig_dst = _pad_pairs(signal_edges, SIG_PAD)
    rnd_src, rnd_dst = _pad_pairs(random_pairs, RND_PAD)

    knn_sum = _knn_hinge_sum(x)
    pair_out = _pair_sums(x, sig_src, sig_dst, rnd_src, rnd_dst)
    sig_sum = jnp.sum(pair_out[:, :LANES])
    rnd_sum = jnp.sum(pair_out[:, LANES:]) - RND_PAD_CORRECTION

    n_sig = signal_edges.shape[1]
    n_rnd = random_pairs.shape[1]
    return (sig_sum / n_sig + knn_sum / (N_NODES * K_NN) + rnd_sum / n_rnd
            ).astype(jnp.float32)
